# baseline (speedup 1.0000x reference)
import functools

import jax
import jax.numpy as jnp
from jax.experimental import pallas as pl
from jax.experimental.pallas import tpu as pltpu
from jax.experimental.pallas import tpu_sc as plsc


def _scatter_body(et_hbm, idx_hbm, out_hbm, et_s, idx_s, sem_et, sem_idx):
    cp_et = pltpu.make_async_copy(et_hbm, et_s, sem_et)
    cp_idx = pltpu.make_async_copy(idx_hbm, idx_s, sem_idx)
    cp_et.start()
    cp_idx.start()
    cp_idx.wait()
    cp_et.wait()
    i0 = idx_s[0]
    i1 = idx_s[1] + 4
    et_s[i0] = et_s[i0] + 1
    et_s[i1] = et_s[i1] + 1
    pltpu.sync_copy(et_s, out_hbm)


_scatter_sc = functools.partial(
    pl.kernel,
    out_type=jax.ShapeDtypeStruct((8,), jnp.int32),
    mesh=plsc.ScalarSubcoreMesh(axis_name="c", num_cores=1),
    scratch_types=[
        pltpu.SMEM((8,), jnp.int32),
        pltpu.SMEM((2,), jnp.int32),
        pltpu.SemaphoreType.DMA,
        pltpu.SemaphoreType.DMA,
    ],
    compiler_params=pltpu.CompilerParams(needs_layout_passes=False),
)(_scatter_body)


def kernel(x, example_tensor, indices):
    del x
    out_flat = _scatter_sc(example_tensor.reshape(8), indices.reshape(2))
    return out_flat.reshape(2, 4)
